# Initial kernel scaffold; baseline (speedup 1.0000x reference)
import jax
import jax.numpy as jnp
from jax.experimental import pallas as pl


def kernel(x, edge_index, W1, W2):
    raise NotImplementedError("write your pallas kernel here")


# speedup vs baseline: 6.6724x; 6.6724x over previous
import functools

import jax
import jax.numpy as jnp
from jax import lax
from jax.experimental import pallas as pl
from jax.experimental.pallas import tpu as pltpu
from jax.experimental.pallas import tpu_sc as plsc

_N = 10000
_E = 320000
_D = 128
_NP = 10240
_NW = 32
_CH = 128
_EPW = 10240
_NCHUNK = _EPW // _CH
_EP = _EPW * _NW
_RPS = _NP // 16

_mesh = plsc.VectorSubcoreMesh(core_axis_name="c", subcore_axis_name="s")


@functools.partial(
    pl.kernel,
    out_type=jax.ShapeDtypeStruct((2, _NP), jnp.float32),
    mesh=_mesh,
    scratch_types=[
        pltpu.VMEM((_CH,), jnp.int32),
        pltpu.VMEM((_CH,), jnp.float32),
        pltpu.VMEM((_RPS,), jnp.float32),
        pltpu.VMEM_SHARED((_NP,), jnp.float32),
    ],
)
def _deg_call(dst_hbm, out_hbm, didx_v, ones_v, zeros_v, deg_sh):
    c = lax.axis_index("c")
    s = lax.axis_index("s")
    wid = s * 2 + c

    def fill_ones(i, carry):
        ones_v[pl.ds(i * 16, 16)] = jnp.full((16,), 1.0, jnp.float32)
        return carry

    lax.fori_loop(0, _CH // 16, fill_ones, 0)

    def fill_zeros(i, carry):
        zeros_v[pl.ds(i * 16, 16)] = jnp.zeros((16,), jnp.float32)
        return carry

    lax.fori_loop(0, _RPS // 16, fill_zeros, 0)

    pltpu.sync_copy(zeros_v, deg_sh.at[pl.ds(s * _RPS, _RPS)])
    plsc.subcore_barrier()

    def body(i, carry):
        base = wid * _EPW + i * _CH
        pltpu.sync_copy(dst_hbm.at[pl.ds(base, _CH)], didx_v)
        pltpu.sync_copy(ones_v, deg_sh.at[didx_v], add=True)
        return carry

    lax.fori_loop(0, _NCHUNK, body, 0)
    plsc.subcore_barrier()
    pltpu.sync_copy(
        deg_sh.at[pl.ds(s * _RPS, _RPS)],
        out_hbm.at[c, pl.ds(s * _RPS, _RPS)],
    )


@functools.partial(
    pl.kernel,
    out_type=jax.ShapeDtypeStruct((2, _NP, _D), jnp.float32),
    mesh=_mesh,
    scratch_types=[
        pltpu.VMEM((_CH,), jnp.int32),
        pltpu.VMEM((_CH,), jnp.int32),
        pltpu.VMEM((_CH, _D), jnp.float32),
        pltpu.SemaphoreType.DMA,
        pltpu.VMEM_SHARED((_NP, _D), jnp.float32),
    ],
)
def _agg_call(h_hbm, src_hbm, dst_hbm, out_hbm, sidx_v, didx_v, rows_v, sem, agg_sh):
    c = lax.axis_index("c")
    s = lax.axis_index("s")
    wid = s * 2 + c

    def zrow(r, carry):
        for j in range(_D // 16):
            rows_v[r, pl.ds(j * 16, 16)] = jnp.zeros((16,), jnp.float32)
        return carry

    lax.fori_loop(0, _CH, zrow, 0)

    for k in range(_RPS // _CH):
        pltpu.sync_copy(rows_v, agg_sh.at[pl.ds(s * _RPS + k * _CH, _CH)])
    plsc.subcore_barrier()

    def body(i, carry):
        base = wid * _EPW + i * _CH
        pltpu.sync_copy(src_hbm.at[pl.ds(base, _CH)], sidx_v)
        pltpu.async_copy(h_hbm.at[sidx_v], rows_v, sem).wait()
        pltpu.sync_copy(dst_hbm.at[pl.ds(base, _CH)], didx_v)
        pltpu.sync_copy(rows_v, agg_sh.at[didx_v], add=True)
        return carry

    lax.fori_loop(0, _NCHUNK, body, 0)
    plsc.subcore_barrier()
    pltpu.sync_copy(
        agg_sh.at[pl.ds(s * _RPS, _RPS)],
        out_hbm.at[c, pl.ds(s * _RPS, _RPS)],
    )


_BLK = 1024
_GRID = _NP // _BLK


def _prep_body(x_ref, dv_ref, w_ref, o_ref):
    o_ref[...] = jnp.dot(
        x_ref[...] * dv_ref[...], w_ref[...],
        preferred_element_type=jnp.float32,
        precision=jax.lax.Precision.HIGHEST,
    )


_prep_call = pl.pallas_call(
    _prep_body,
    grid=(_GRID,),
    in_specs=[
        pl.BlockSpec((_BLK, _D), lambda i: (i, 0)),
        pl.BlockSpec((_BLK, _D), lambda i: (i, 0)),
        pl.BlockSpec((_D, _D), lambda i: (0, 0)),
    ],
    out_specs=pl.BlockSpec((_BLK, _D), lambda i: (i, 0)),
    out_shape=jax.ShapeDtypeStruct((_NP, _D), jnp.float32),
)


def _post1_body(p_ref, dv_ref, w_ref, h_ref, g_ref):
    dv = dv_ref[...]
    h = jnp.maximum((p_ref[0] + p_ref[1]) * dv, 0.0)
    h_ref[...] = h
    g_ref[...] = jnp.dot(
        h * dv, w_ref[...],
        preferred_element_type=jnp.float32,
        precision=jax.lax.Precision.HIGHEST,
    )


_post1_call = pl.pallas_call(
    _post1_body,
    grid=(_GRID,),
    in_specs=[
        pl.BlockSpec((2, _BLK, _D), lambda i: (0, i, 0)),
        pl.BlockSpec((_BLK, _D), lambda i: (i, 0)),
        pl.BlockSpec((_D, _D), lambda i: (0, 0)),
    ],
    out_specs=[
        pl.BlockSpec((_BLK, _D), lambda i: (i, 0)),
        pl.BlockSpec((_BLK, _D), lambda i: (i, 0)),
    ],
    out_shape=[
        jax.ShapeDtypeStruct((_NP, _D), jnp.float32),
        jax.ShapeDtypeStruct((_NP, _D), jnp.float32),
    ],
)


def _post2_body(p_ref, dv_ref, h_ref):
    h_ref[...] = jnp.maximum((p_ref[0] + p_ref[1]) * dv_ref[...], 0.0)


_post2_call = pl.pallas_call(
    _post2_body,
    grid=(_GRID,),
    in_specs=[
        pl.BlockSpec((2, _BLK, _D), lambda i: (0, i, 0)),
        pl.BlockSpec((_BLK, _D), lambda i: (i, 0)),
    ],
    out_specs=pl.BlockSpec((_BLK, _D), lambda i: (i, 0)),
    out_shape=jax.ShapeDtypeStruct((_NP, _D), jnp.float32),
)


def kernel(x, edge_index, W1, W2):
    src = edge_index[0]
    dst = edge_index[1]

    x_p = jnp.zeros((_NP, _D), jnp.float32).at[:_N].set(x)
    pad = jnp.full((_EP - _E,), _NP - 1, jnp.int32)
    src_p = jnp.concatenate([src, pad])
    dst_p = jnp.concatenate([dst, pad])

    degp = _deg_call(dst_p)
    dinv = jax.lax.rsqrt(jnp.maximum(degp[0] + degp[1], 1.0))
    dinv_mat = jnp.broadcast_to(dinv[:, None], (_NP, _D))

    g1 = _prep_call(x_p, dinv_mat, W1)
    P1 = _agg_call(g1, src_p, dst_p)
    h1, g2 = _post1_call(P1, dinv_mat, W2)
    P2 = _agg_call(g2, src_p, dst_p)
    h2 = _post2_call(P2, dinv_mat)

    return jnp.stack([x, h1[:_N], h2[:_N]], axis=0)
